# baseline (speedup 1.0000x reference)
import functools

import jax
import jax.numpy as jnp
from jax import lax
from jax.experimental import pallas as pl
from jax.experimental.pallas import tpu as pltpu
from jax.experimental.pallas import tpu_sc as plsc

_NC, _NS = 2, 16
_NW = _NC * _NS


@functools.lru_cache(maxsize=None)
def _build_gather(b, h, v, d):
    tr_n = h // 8
    tb_n = b // 128
    dhi_n = d // 8
    units = h
    assert tb_n == _NW and h % 8 == 0 and b % 128 == 0 and d == 32

    mesh = plsc.VectorSubcoreMesh(core_axis_name="c", subcore_axis_name="s")

    @functools.partial(
        pl.kernel,
        mesh=mesh,
        out_type=jax.ShapeDtypeStruct((h, dhi_n, tb_n, 8 * 128), jnp.float32),
        scratch_types=[
            pltpu.VMEM((tr_n, 8, 128), jnp.int32),
            pltpu.VMEM((2, 128, d), jnp.float32),
            pltpu.VMEM((2, d * 128), jnp.float32),
            [pltpu.SemaphoreType.DMA] * 2,
            [pltpu.SemaphoreType.DMA] * 2,
        ],
        compiler_params=pltpu.CompilerParams(
            use_tc_tiling_on_sc=False, needs_layout_passes=False
        ),
    )
    def gather_kernel(idx_hbm, table_hbm, out_hbm, idx_v, rows_v, t_v, gsem, ssem):
        tb = lax.axis_index("s") * _NC + lax.axis_index("c")
        for tr in range(tr_n):
            pltpu.sync_copy(idx_hbm.at[tr, tb], idx_v.at[tr])

        base0 = lax.iota(jnp.int32, 16) * 128
        base1 = base0 + 16 * 128

        def start_gather(u, buf):
            pltpu.async_copy(
                table_hbm.at[idx_v.at[u >> 3, u & 7]], rows_v.at[buf], gsem[buf]
            )

        def wait_gather(u, buf):
            pltpu.make_async_copy(
                table_hbm.at[idx_v.at[u >> 3, u & 7]], rows_v.at[buf], gsem[buf]
            ).wait()

        def transpose(buf):
            def grp(g, carry):
                for u16 in range(16):
                    bb = g * 16 + u16
                    v0 = rows_v[buf, bb, pl.ds(0, 16)]
                    v1 = rows_v[buf, bb, pl.ds(16, 16)]
                    plsc.store_scatter(t_v.at[buf], [base0 + bb], v0)
                    plsc.store_scatter(t_v.at[buf], [base1 + bb], v1)
                return carry

            lax.fori_loop(0, 8, grp, 0)

        def start_stores(u, buf):
            for j in range(dhi_n):
                pltpu.async_copy(
                    t_v.at[buf, pl.ds(j * 1024, 1024)],
                    out_hbm.at[u, j, tb],
                    ssem[buf],
                )

        def wait_stores(u, buf):
            for j in range(dhi_n):
                pltpu.make_async_copy(
                    t_v.at[buf, pl.ds(j * 1024, 1024)],
                    out_hbm.at[u, j, tb],
                    ssem[buf],
                ).wait()

        for p in range(2):
            start_gather(jnp.int32(p), p)
        for p in range(2):
            u = jnp.int32(p)
            wait_gather(u, p)
            transpose(p)
            start_gather(u + 2, p)
            start_stores(u, p)

        def steady(j, carry):
            for p in range(2):
                u = j * 2 + p
                wait_stores(u, p)
                wait_gather(u, p)
                transpose(p)
                start_gather(u + 2, p)
                start_stores(u, p)
            return carry

        lax.fori_loop(1, units // 2 - 1, steady, 0)

        for p in range(2):
            u = jnp.int32(units - 2 + p)
            wait_stores(u, p)
            wait_gather(u, p)
            transpose(p)
            start_stores(u, p)
        for p in range(2):
            wait_stores(jnp.int32(units - 2 + p), p)

    return gather_kernel


def kernel(item_ids, item_emb_weight):
    b, h = item_ids.shape
    v, d = item_emb_weight.shape
    fn = _build_gather(b, h, v, d)
    ids4 = (
        item_ids.astype(jnp.int32)
        .T.reshape(h // 8, 8, b // 128, 128)
        .transpose(0, 2, 1, 3)
    )
    out4 = fn(ids4, item_emb_weight)
    return (
        out4.reshape(h, d // 8, b // 128, 8, 128)
        .transpose(2, 4, 0, 1, 3)
        .reshape(b, h, d)
    )


# speedup vs baseline: 1.5010x; 1.5010x over previous
import functools

import jax
import jax.numpy as jnp
from jax import lax
from jax.experimental import pallas as pl
from jax.experimental.pallas import tpu as pltpu
from jax.experimental.pallas import tpu_sc as plsc

_NC, _NS = 2, 16
_NW = _NC * _NS


@functools.lru_cache(maxsize=None)
def _build_gather(b, h, v, d):
    tr_n = h // 8
    tb_n = b // 128
    dhi_n = d // 8
    units = h
    assert tb_n == _NW and h % 8 == 0 and b % 128 == 0 and d == 32

    mesh = plsc.VectorSubcoreMesh(core_axis_name="c", subcore_axis_name="s")

    @functools.partial(
        pl.kernel,
        mesh=mesh,
        out_type=jax.ShapeDtypeStruct((h, dhi_n, tb_n, 8, 128), jnp.float32),
        scratch_types=[
            pltpu.VMEM((tr_n, 8, 128), jnp.int32),
            pltpu.VMEM((2, 128, d), jnp.float32),
            pltpu.VMEM((2, d, 129), jnp.float32),
            [pltpu.SemaphoreType.DMA] * 2,
            [pltpu.SemaphoreType.DMA] * 2,
        ],
        compiler_params=pltpu.CompilerParams(
            use_tc_tiling_on_sc=False, needs_layout_passes=False
        ),
    )
    def gather_kernel(idx_hbm, table_hbm, out_hbm, idx_v, rows_v, t_v, gsem, ssem):
        tb = lax.axis_index("s") * _NC + lax.axis_index("c")
        for tr in range(tr_n):
            pltpu.sync_copy(idx_hbm.at[tr, tb], idx_v.at[tr])

        dvec0 = lax.iota(jnp.int32, 16)
        dvec1 = dvec0 + 16

        def start_gather(u, buf):
            pltpu.async_copy(
                table_hbm.at[idx_v.at[u >> 3, u & 7]], rows_v.at[buf], gsem[buf]
            )

        def wait_gather(u, buf):
            pltpu.make_async_copy(
                table_hbm.at[idx_v.at[u >> 3, u & 7]], rows_v.at[buf], gsem[buf]
            ).wait()

        def transpose(buf):
            def grp(g, carry):
                for u16 in range(16):
                    bb = g * 16 + u16
                    bvec = jnp.full((16,), 0, jnp.int32) + bb
                    v0 = rows_v[buf, bb, pl.ds(0, 16)]
                    v1 = rows_v[buf, bb, pl.ds(16, 16)]
                    plsc.store_scatter(t_v.at[buf], [dvec0, bvec], v0)
                    plsc.store_scatter(t_v.at[buf], [dvec1, bvec], v1)
                return carry

            lax.fori_loop(0, 8, grp, 0)

        def start_stores(u, buf):
            for j in range(dhi_n):
                pltpu.async_copy(
                    t_v.at[buf, pl.ds(j * 8, 8), pl.ds(0, 128)],
                    out_hbm.at[u, j, tb],
                    ssem[buf],
                )

        def wait_stores(u, buf):
            for j in range(dhi_n):
                pltpu.make_async_copy(
                    t_v.at[buf, pl.ds(j * 8, 8), pl.ds(0, 128)],
                    out_hbm.at[u, j, tb],
                    ssem[buf],
                ).wait()

        for p in range(2):
            start_gather(jnp.int32(p), p)
        for p in range(2):
            u = jnp.int32(p)
            wait_gather(u, p)
            transpose(p)
            start_gather(u + 2, p)
            start_stores(u, p)

        def steady(j, carry):
            for p in range(2):
                u = j * 2 + p
                wait_stores(u, p)
                wait_gather(u, p)
                transpose(p)
                start_gather(u + 2, p)
                start_stores(u, p)
            return carry

        lax.fori_loop(1, units // 2 - 1, steady, 0)

        for p in range(2):
            u = jnp.int32(units - 2 + p)
            wait_stores(u, p)
            wait_gather(u, p)
            transpose(p)
            start_stores(u, p)
        for p in range(2):
            wait_stores(jnp.int32(units - 2 + p), p)

    return gather_kernel


def kernel(item_ids, item_emb_weight):
    b, h = item_ids.shape
    v, d = item_emb_weight.shape
    fn = _build_gather(b, h, v, d)
    ids4 = (
        item_ids.astype(jnp.int32)
        .T.reshape(h // 8, 8, b // 128, 128)
        .transpose(0, 2, 1, 3)
    )
    out5 = fn(ids4, item_emb_weight)
    return out5.transpose(2, 4, 0, 1, 3).reshape(b, h, d)


# speedup vs baseline: 1.5018x; 1.0006x over previous
import functools

import jax
import jax.numpy as jnp
from jax import lax
from jax.experimental import pallas as pl
from jax.experimental.pallas import tpu as pltpu
from jax.experimental.pallas import tpu_sc as plsc

_NC, _NS = 2, 16
_NW = _NC * _NS


@functools.lru_cache(maxsize=None)
def _build_gather(b, h, v, d):
    tr_n = h // 8
    tb_n = b // 128
    dhi_n = d // 8
    units = h
    assert tb_n == _NW and h % 8 == 0 and b % 128 == 0 and d == 32

    mesh = plsc.VectorSubcoreMesh(core_axis_name="c", subcore_axis_name="s")

    @functools.partial(
        pl.kernel,
        mesh=mesh,
        out_type=jax.ShapeDtypeStruct((h, dhi_n, tb_n, 8, 128), jnp.float32),
        scratch_types=[
            pltpu.VMEM((tr_n, 8, 128), jnp.int32),
            pltpu.VMEM((2, 128, d), jnp.float32),
            pltpu.VMEM((2, d, 129), jnp.float32),
            [pltpu.SemaphoreType.DMA] * 2,
            [pltpu.SemaphoreType.DMA] * 2,
        ],
        compiler_params=pltpu.CompilerParams(
            use_tc_tiling_on_sc=False, needs_layout_passes=False
        ),
    )
    def gather_kernel(idx_hbm, table_hbm, out_hbm, idx_v, rows_v, t_v, gsem, ssem):
        tb = lax.axis_index("s") * _NC + lax.axis_index("c")
        for tr in range(tr_n):
            pltpu.sync_copy(idx_hbm.at[tr, tb], idx_v.at[tr])

        dvec0 = lax.iota(jnp.int32, 16)
        dvec1 = dvec0 + 16

        def start_gather(u, buf):
            pltpu.async_copy(
                table_hbm.at[idx_v.at[u >> 3, u & 7]], rows_v.at[buf], gsem[buf]
            )

        def wait_gather(u, buf):
            pltpu.make_async_copy(
                table_hbm.at[idx_v.at[u >> 3, u & 7]], rows_v.at[buf], gsem[buf]
            ).wait()

        def transpose(buf):
            def grp(g, carry):
                for u16 in range(16):
                    bb = g * 16 + u16
                    bvec = jnp.full((16,), 0, jnp.int32) + bb
                    v0 = rows_v[buf, bb, pl.ds(0, 16)]
                    v1 = rows_v[buf, bb, pl.ds(16, 16)]
                    plsc.store_scatter(t_v.at[buf], [dvec0, bvec], v0)
                    plsc.store_scatter(t_v.at[buf], [dvec1, bvec], v1)
                return carry

            lax.fori_loop(0, 8, grp, 0)

        def start_stores(u, buf):
            for j in range(dhi_n):
                pltpu.async_copy(
                    t_v.at[buf, pl.ds(j * 8, 8), pl.ds(0, 128)],
                    out_hbm.at[u, j, tb],
                    ssem[buf],
                )

        def wait_stores(u, buf):
            for j in range(dhi_n):
                pltpu.make_async_copy(
                    t_v.at[buf, pl.ds(j * 8, 8), pl.ds(0, 128)],
                    out_hbm.at[u, j, tb],
                    ssem[buf],
                ).wait()

        for p in range(2):
            start_gather(jnp.int32(p), p)
        for p in range(2):
            u = jnp.int32(p)
            wait_gather(u, p)
            transpose(p)
            start_gather(u + 2, p)
            start_stores(u, p)

        def steady(j, carry):
            for p in range(2):
                u = j * 2 + p
                wait_stores(u, p)
                wait_gather(u, p)
                transpose(p)
                start_gather(u + 2, p)
                start_stores(u, p)
            return carry

        lax.fori_loop(1, units // 2 - 1, steady, 0)

        for p in range(2):
            u = jnp.int32(units - 2 + p)
            wait_stores(u, p)
            wait_gather(u, p)
            transpose(p)
            start_stores(u, p)
        for p in range(2):
            wait_stores(jnp.int32(units - 2 + p), p)

    return gather_kernel


def kernel(item_ids, item_emb_weight):
    b, h = item_ids.shape
    v, d = item_emb_weight.shape
    v_eff = (v - 1) if (v - 1) % 32 == 0 else v
    fn = _build_gather(b, h, v_eff, d)
    ids4 = (
        item_ids.astype(jnp.int32)
        .T.reshape(h // 8, 8, b // 128, 128)
        .transpose(0, 2, 1, 3)
    )
    out5 = fn(ids4, item_emb_weight[:v_eff])
    return out5.transpose(2, 4, 0, 1, 3).reshape(b, h, d)


# speedup vs baseline: 1.5504x; 1.0324x over previous
import functools

import jax
import jax.numpy as jnp
from jax import lax
from jax.experimental import pallas as pl
from jax.experimental.pallas import tpu as pltpu
from jax.experimental.pallas import tpu_sc as plsc

_NC, _NS = 2, 16
_NW = _NC * _NS


@functools.lru_cache(maxsize=None)
def _build_gather(b, h, v, d):
    tr_n = h // 8
    tb_n = b // 128
    dhi_n = d // 8
    units = h
    assert tb_n == _NW and h % 8 == 0 and b % 128 == 0 and d == 32

    mesh = plsc.VectorSubcoreMesh(core_axis_name="c", subcore_axis_name="s")

    @functools.partial(
        pl.kernel,
        mesh=mesh,
        out_type=jax.ShapeDtypeStruct((h, dhi_n, tb_n, 8, 128), jnp.float32),
        scratch_types=[
            pltpu.VMEM((tr_n, 8, 128), jnp.int32),
            pltpu.VMEM((2, 128, d), jnp.float32),
            pltpu.VMEM((2, d, 129), jnp.float32),
            [pltpu.SemaphoreType.DMA] * 2,
            [pltpu.SemaphoreType.DMA] * 2,
        ],
        compiler_params=pltpu.CompilerParams(
            use_tc_tiling_on_sc=False, needs_layout_passes=False
        ),
    )
    def gather_kernel(idx_hbm, table_hbm, out_hbm, idx_v, rows_v, t_v, gsem, ssem):
        tb = lax.axis_index("s") * _NC + lax.axis_index("c")
        for tr in range(tr_n):
            pltpu.sync_copy(idx_hbm.at[tr, tb], idx_v.at[tr])

        dvec0 = lax.iota(jnp.int32, 16)
        dvec1 = dvec0 + 16

        def start_gather(u, buf):
            pltpu.async_copy(
                table_hbm.at[idx_v.at[u >> 3, u & 7]], rows_v.at[buf], gsem[buf]
            )

        def wait_gather(u, buf):
            pltpu.make_async_copy(
                table_hbm.at[idx_v.at[u >> 3, u & 7]], rows_v.at[buf], gsem[buf]
            ).wait()

        def transpose(buf):
            def grp(g, carry):
                for u16 in range(16):
                    bb = g * 16 + u16
                    bvec = jnp.full((16,), 0, jnp.int32) + bb
                    v0 = rows_v[buf, bb, pl.ds(0, 16)]
                    v1 = rows_v[buf, bb, pl.ds(16, 16)]
                    plsc.store_scatter(t_v.at[buf], [dvec0, bvec], v0)
                    plsc.store_scatter(t_v.at[buf], [dvec1, bvec], v1)
                return carry

            lax.fori_loop(0, 8, grp, 0)

        def start_stores(u, buf):
            for j in range(dhi_n):
                pltpu.async_copy(
                    t_v.at[buf, pl.ds(j * 8, 8), pl.ds(0, 128)],
                    out_hbm.at[u, j, tb],
                    ssem[buf],
                )

        def wait_stores(u, buf):
            for j in range(dhi_n):
                pltpu.make_async_copy(
                    t_v.at[buf, pl.ds(j * 8, 8), pl.ds(0, 128)],
                    out_hbm.at[u, j, tb],
                    ssem[buf],
                ).wait()

        for p in range(2):
            start_gather(jnp.int32(p), p)
        for p in range(2):
            u = jnp.int32(p)
            wait_gather(u, p)
            transpose(p)
            start_gather(u + 2, p)
            start_stores(u, p)

        def steady(j, carry):
            for p in range(2):
                u = j * 2 + p
                wait_stores(u, p)
                wait_gather(u, p)
                transpose(p)
                start_gather(u + 2, p)
                start_stores(u, p)
            return carry

        lax.fori_loop(1, units // 2 - 1, steady, 0)

        for p in range(2):
            u = jnp.int32(units - 2 + p)
            wait_stores(u, p)
            wait_gather(u, p)
            transpose(p)
            start_stores(u, p)
        for p in range(2):
            wait_stores(jnp.int32(units - 2 + p), p)

    return gather_kernel


@functools.lru_cache(maxsize=None)
def _build_relayout(v_main, d, v_eff):
    vhi_n = v_main // 128
    lv_n = v_eff - v_main
    per_w = vhi_n // _NW
    extra = vhi_n % _NW

    mesh = plsc.VectorSubcoreMesh(core_axis_name="c", subcore_axis_name="s")

    @functools.partial(
        pl.kernel,
        mesh=mesh,
        out_type=jax.ShapeDtypeStruct((v_eff, d), jnp.float32),
        scratch_types=[
            pltpu.VMEM((2, d // 8, 8, 128), jnp.float32),
            pltpu.VMEM((2, 128, 33), jnp.float32),
            pltpu.VMEM((d, 64), jnp.float32),
            [pltpu.SemaphoreType.DMA] * 2,
            [pltpu.SemaphoreType.DMA] * 2,
        ],
        compiler_params=pltpu.CompilerParams(
            use_tc_tiling_on_sc=False, needs_layout_passes=False
        ),
    )
    def relayout_kernel(nat_hbm, lv_hbm, out_hbm, in_v, t_v, lv_v, gsem, ssem):
        wid = lax.axis_index("s") * _NC + lax.axis_index("c")
        start = wid * per_w + jnp.minimum(wid, extra)
        count = per_w + jnp.where(wid < extra, 1, 0)

        iv = lax.iota(jnp.int32, 16)
        bases = [iv + q * 16 for q in range(8)]
        dvecs = [jnp.full((16,), dd, jnp.int32) for dd in range(d)]

        def start_in(vhi, buf):
            pltpu.async_copy(
                nat_hbm.at[pl.ds(0, d // 8), vhi], in_v.at[buf], gsem[buf]
            )

        def wait_in(vhi, buf):
            pltpu.make_async_copy(
                nat_hbm.at[pl.ds(0, d // 8), vhi], in_v.at[buf], gsem[buf]
            ).wait()

        def transpose(buf):
            for dhi in range(d // 8):
                for dlo in range(8):
                    dd = dhi * 8 + dlo
                    for q in range(8):
                        vv = in_v[buf, dhi, dlo, pl.ds(q * 16, 16)]
                        plsc.store_scatter(
                            t_v.at[buf], [bases[q], dvecs[dd]], vv
                        )

        def start_out(vhi, buf):
            pltpu.async_copy(
                t_v.at[buf, pl.ds(0, 128), pl.ds(0, d)],
                out_hbm.at[pl.ds(vhi * 128, 128)],
                ssem[buf],
            )

        def wait_out(vhi, buf):
            pltpu.make_async_copy(
                t_v.at[buf, pl.ds(0, 128), pl.ds(0, d)],
                out_hbm.at[pl.ds(vhi * 128, 128)],
                ssem[buf],
            ).wait()

        for p in range(2):
            @pl.when(count > p)
            def _prime(p=p):
                start_in(start + p, p)

        def body(k, carry):
            for p in range(2):
                kk = k * 2 + p
                @pl.when(count > kk)
                def _do(kk=kk, p=p):
                    vhi = start + kk
                    @pl.when(kk >= 2)
                    def _drain():
                        wait_out(vhi - 2, p)
                    wait_in(vhi, p)
                    transpose(p)
                    @pl.when(count > kk + 2)
                    def _next():
                        start_in(vhi + 2, p)
                    start_out(vhi, p)
            return carry

        lax.fori_loop(0, per_w // 2 + 2, body, 0)
        for p in range(2):
            @pl.when(count > p)
            def _fin(p=p):
                rem = count - 1
                kk = jnp.where((rem & 1) == p, rem, rem - 1)
                @pl.when(kk >= 0)
                def _w():
                    wait_out(start + kk, p)

        @pl.when(wid == 0)
        def _leftover():
            pltpu.sync_copy(lv_hbm, lv_v)
            for dd in range(d):
                for q in range(4):
                    vv = lv_v[dd, pl.ds(q * 16, 16)]
                    plsc.store_scatter(t_v.at[0], [bases[q], dvecs[dd]], vv)
            pltpu.sync_copy(
                t_v.at[0, pl.ds(0, lv_n), pl.ds(0, d)],
                out_hbm.at[pl.ds(v_main, lv_n)],
            )

    return relayout_kernel


def kernel(item_ids, item_emb_weight):
    b, h = item_ids.shape
    v, d = item_emb_weight.shape
    v_eff = v - 1
    v_main = (v_eff // 128) * 128
    ids4 = (
        item_ids.astype(jnp.int32)
        .T.reshape(h // 8, 8, b // 128, 128)
        .transpose(0, 2, 1, 3)
    )
    tt = item_emb_weight.T
    nat4 = tt[:, :v_main].reshape(d // 8, 8, v_main // 128, 128).transpose(0, 2, 1, 3)
    lv = tt[:, v_main:v_eff]
    table2 = _build_relayout(v_main, d, v_eff)(nat4, lv)
    fn = _build_gather(b, h, v_eff, d)
    out5 = fn(ids4, table2)
    return out5.transpose(2, 4, 0, 1, 3).reshape(b, h, d)


# speedup vs baseline: 1.9346x; 1.2478x over previous
import functools

import jax
import jax.numpy as jnp
from jax import lax
from jax.experimental import pallas as pl
from jax.experimental.pallas import tpu as pltpu
from jax.experimental.pallas import tpu_sc as plsc

_NC, _NS = 2, 16
_NW = _NC * _NS


@functools.lru_cache(maxsize=None)
def _build_gather(b, h, v, d):
    tr_n = h // 8
    tb_n = b // 128
    dhi_n = d // 8
    units = h
    assert tb_n == _NW and h % 8 == 0 and b % 128 == 0 and d == 32

    mesh = plsc.VectorSubcoreMesh(core_axis_name="c", subcore_axis_name="s")

    @functools.partial(
        pl.kernel,
        mesh=mesh,
        out_type=jax.ShapeDtypeStruct((h, dhi_n, tb_n, 8, 128), jnp.float32),
        scratch_types=[
            pltpu.VMEM((tr_n, 8, 128), jnp.int32),
            pltpu.VMEM((2, 128, d), jnp.float32),
            pltpu.VMEM((2, d, 129), jnp.float32),
            [pltpu.SemaphoreType.DMA] * 2,
            [pltpu.SemaphoreType.DMA] * 2,
        ],
        compiler_params=pltpu.CompilerParams(
            use_tc_tiling_on_sc=False, needs_layout_passes=False
        ),
    )
    def gather_kernel(idx_hbm, table_hbm, out_hbm, idx_v, rows_v, t_v, gsem, ssem):
        tb = lax.axis_index("s") * _NC + lax.axis_index("c")
        for tr in range(tr_n):
            pltpu.sync_copy(idx_hbm.at[tr, tb], idx_v.at[tr])

        dvec0 = lax.iota(jnp.int32, 16)
        dvec1 = dvec0 + 16

        def start_gather(u, buf):
            pltpu.async_copy(
                table_hbm.at[idx_v.at[u >> 3, u & 7]], rows_v.at[buf], gsem[buf]
            )

        def wait_gather(u, buf):
            pltpu.make_async_copy(
                table_hbm.at[idx_v.at[u >> 3, u & 7]], rows_v.at[buf], gsem[buf]
            ).wait()

        def transpose(buf):
            def grp(g, carry):
                for u16 in range(16):
                    bb = g * 16 + u16
                    bvec = jnp.full((16,), 0, jnp.int32) + bb
                    v0 = rows_v[buf, bb, pl.ds(0, 16)]
                    v1 = rows_v[buf, bb, pl.ds(16, 16)]
                    plsc.store_scatter(t_v.at[buf], [dvec0, bvec], v0)
                    plsc.store_scatter(t_v.at[buf], [dvec1, bvec], v1)
                return carry

            lax.fori_loop(0, 8, grp, 0)

        def start_stores(u, buf):
            for j in range(dhi_n):
                pltpu.async_copy(
                    t_v.at[buf, pl.ds(j * 8, 8), pl.ds(0, 128)],
                    out_hbm.at[u, j, tb],
                    ssem[buf],
                )

        def wait_stores(u, buf):
            for j in range(dhi_n):
                pltpu.make_async_copy(
                    t_v.at[buf, pl.ds(j * 8, 8), pl.ds(0, 128)],
                    out_hbm.at[u, j, tb],
                    ssem[buf],
                ).wait()

        for p in range(2):
            start_gather(jnp.int32(p), p)
        for p in range(2):
            u = jnp.int32(p)
            wait_gather(u, p)
            transpose(p)
            start_gather(u + 2, p)
            start_stores(u, p)

        def steady(j, carry):
            for p in range(2):
                u = j * 2 + p
                wait_stores(u, p)
                wait_gather(u, p)
                transpose(p)
                start_gather(u + 2, p)
                start_stores(u, p)
            return carry

        lax.fori_loop(1, units // 2 - 1, steady, 0)

        for p in range(2):
            u = jnp.int32(units - 2 + p)
            wait_stores(u, p)
            wait_gather(u, p)
            transpose(p)
            start_stores(u, p)
        for p in range(2):
            wait_stores(jnp.int32(units - 2 + p), p)

    return gather_kernel


@functools.lru_cache(maxsize=None)
def _build_relayout(v_main, d, v_eff):
    vhi_n = v_main // 128
    lv_n = v_eff - v_main
    per_w = vhi_n // _NW
    extra = vhi_n % _NW

    mesh = plsc.VectorSubcoreMesh(core_axis_name="c", subcore_axis_name="s")

    @functools.partial(
        pl.kernel,
        mesh=mesh,
        out_type=jax.ShapeDtypeStruct((v_eff, d), jnp.float32),
        scratch_types=[
            pltpu.VMEM((2, d // 8, 8, 133), jnp.float32),
            pltpu.VMEM((2, 128, d), jnp.float32),
            pltpu.VMEM((d, 69), jnp.float32),
            [pltpu.SemaphoreType.DMA] * 2,
            [pltpu.SemaphoreType.DMA] * 2,
        ],
        compiler_params=pltpu.CompilerParams(
            use_tc_tiling_on_sc=False, needs_layout_passes=False
        ),
    )
    def relayout_kernel(nat_hbm, lv_hbm, out_hbm, in_v, t_v, lv_v, gsem, ssem):
        wid = lax.axis_index("s") * _NC + lax.axis_index("c")
        start = wid * per_w + jnp.minimum(wid, extra)
        count = per_w + jnp.where(wid < extra, 1, 0)

        iv = lax.iota(jnp.int32, 16)
        i0a, i1a = iv >> 3, iv & 7
        i0b = i0a + 2

        def start_in(vhi, buf):
            pltpu.async_copy(
                nat_hbm.at[pl.ds(0, d // 8), vhi],
                in_v.at[buf, pl.ds(0, d // 8), pl.ds(0, 8), pl.ds(0, 128)],
                gsem[buf],
            )

        def wait_in(vhi, buf):
            pltpu.make_async_copy(
                nat_hbm.at[pl.ds(0, d // 8), vhi],
                in_v.at[buf, pl.ds(0, d // 8), pl.ds(0, 8), pl.ds(0, 128)],
                gsem[buf],
            ).wait()

        def transpose(buf):
            def grp(g, carry):
                for u16 in range(16):
                    vlo = g * 16 + u16
                    bv = jnp.full((16,), 0, jnp.int32) + vlo
                    g0 = plsc.load_gather(in_v.at[buf], [i0a, i1a, bv])
                    g1 = plsc.load_gather(in_v.at[buf], [i0b, i1a, bv])
                    t_v[buf, vlo, pl.ds(0, 16)] = g0
                    t_v[buf, vlo, pl.ds(16, 16)] = g1
                return carry

            lax.fori_loop(0, 8, grp, 0)

        def start_out(vhi, buf):
            pltpu.async_copy(
                t_v.at[buf], out_hbm.at[pl.ds(vhi * 128, 128)], ssem[buf]
            )

        def wait_out(vhi, buf):
            pltpu.make_async_copy(
                t_v.at[buf], out_hbm.at[pl.ds(vhi * 128, 128)], ssem[buf]
            ).wait()

        for p in range(2):
            @pl.when(count > p)
            def _prime(p=p):
                start_in(start + p, p)

        def body(k, carry):
            for p in range(2):
                kk = k * 2 + p
                @pl.when(count > kk)
                def _do(kk=kk, p=p):
                    vhi = start + kk
                    @pl.when(kk >= 2)
                    def _drain():
                        wait_out(vhi - 2, p)
                    wait_in(vhi, p)
                    transpose(p)
                    @pl.when(count > kk + 2)
                    def _next():
                        start_in(vhi + 2, p)
                    start_out(vhi, p)
            return carry

        lax.fori_loop(0, per_w // 2 + 2, body, 0)
        for p in range(2):
            @pl.when(count > p)
            def _fin(p=p):
                rem = count - 1
                kk = jnp.where((rem & 1) == p, rem, rem - 1)
                @pl.when(kk >= 0)
                def _w():
                    wait_out(start + kk, p)

        @pl.when(wid == 0)
        def _leftover():
            pltpu.sync_copy(lv_hbm, lv_v.at[pl.ds(0, d), pl.ds(0, lv_n)])
            for vl in range(lv_n):
                bv = jnp.full((16,), 0, jnp.int32) + vl
                g0 = plsc.load_gather(lv_v, [iv, bv])
                g1 = plsc.load_gather(lv_v, [iv + 16, bv])
                t_v[0, vl, pl.ds(0, 16)] = g0
                t_v[0, vl, pl.ds(16, 16)] = g1
            pltpu.sync_copy(
                t_v.at[0, pl.ds(0, lv_n), pl.ds(0, d)],
                out_hbm.at[pl.ds(v_main, lv_n)],
            )

    return relayout_kernel


def kernel(item_ids, item_emb_weight):
    b, h = item_ids.shape
    v, d = item_emb_weight.shape
    v_eff = v - 1
    v_main = (v_eff // 128) * 128
    ids4 = (
        item_ids.astype(jnp.int32)
        .T.reshape(h // 8, 8, b // 128, 128)
        .transpose(0, 2, 1, 3)
    )
    tt = item_emb_weight.T
    nat4 = tt[:, :v_main].reshape(d // 8, 8, v_main // 128, 128).transpose(0, 2, 1, 3)
    lv = tt[:, v_main:v_eff]
    table2 = _build_relayout(v_main, d, v_eff)(nat4, lv)
    fn = _build_gather(b, h, v_eff, d)
    out5 = fn(ids4, table2)
    return out5.transpose(2, 4, 0, 1, 3).reshape(b, h, d)
